# baseline (speedup 1.0000x reference)
import functools
import math

import jax
import jax.numpy as jnp
from jax import lax
from jax.experimental import pallas as pl
from jax.experimental.pallas import tpu as pltpu

_LOG2E = 1.4426950408889634
_LN2 = 0.6931471805599453


def _round_up(x, m):
    return (x + m - 1) // m * m


def _arcface_body(emb_ref, w_ref, lab_ref, l_ref, t_ref, embn_scr,
                  *, s2, num_classes, tile_c, nc, mask_cols):
    c = pl.program_id(0)

    @pl.when(c == 0)
    def _init():
        emb = emb_ref[...]
        inv = lax.rsqrt(jnp.maximum(jnp.sum(emb * emb, axis=1, keepdims=True),
                                    1e-24))
        embn_scr[...] = (emb * inv).astype(jnp.bfloat16)
        l_ref[...] = jnp.zeros(l_ref.shape, jnp.float32)
        t_ref[...] = jnp.zeros(t_ref.shape, jnp.float32)

    w = w_ref[...]
    inv_w = lax.rsqrt(jnp.maximum(jnp.sum(w * w, axis=1, keepdims=True), 1e-24))
    w_s = (w * (inv_w * s2)).astype(jnp.bfloat16)

    logits2 = lax.dot_general(
        embn_scr[...], w_s,
        dimension_numbers=(((1,), (1,)), ((), ())),
        preferred_element_type=jnp.float32)

    e = jnp.exp2(logits2)
    col = jax.lax.broadcasted_iota(jnp.int32, logits2.shape, 1)
    lab_loc = lab_ref[...] - c * tile_c
    one_hot = col == lab_loc
    if mask_cols:
        e = jnp.where(col + c * tile_c < num_classes, e, 0.0)
    l_ref[...] += jnp.sum(e, axis=1, keepdims=True)
    t_ref[...] += jnp.sum(jnp.where(one_hot, e, 0.0), axis=1, keepdims=True)


def _arcface_loss(embeddings, weight, labels, margin=0.2, scale=30.0):
    B, D = embeddings.shape
    C, D2 = weight.shape
    assert D == D2

    TILE_C = 2048
    B_pad = _round_up(B, 8)
    C_pad = _round_up(C, TILE_C)
    if B_pad != B:
        embeddings = jnp.pad(embeddings, ((0, B_pad - B), (0, 0)))
        labels = jnp.pad(labels, (0, B_pad - B))
    if C_pad != C:
        weight = jnp.pad(weight, ((0, C_pad - C), (0, 0)))
    nc = C_pad // TILE_C
    labels2d = labels.astype(jnp.int32).reshape(B_pad, 1)

    s2 = scale * _LOG2E
    body = functools.partial(
        _arcface_body, s2=s2, num_classes=C, tile_c=TILE_C, nc=nc,
        mask_cols=(C_pad != C))

    l_parts, t_parts = pl.pallas_call(
        body,
        out_shape=(jax.ShapeDtypeStruct((B_pad, 1), jnp.float32),
                   jax.ShapeDtypeStruct((B_pad, 1), jnp.float32)),
        grid=(nc,),
        in_specs=[
            pl.BlockSpec((B_pad, D), lambda c: (0, 0)),
            pl.BlockSpec((TILE_C, D), lambda c: (c, 0)),
            pl.BlockSpec((B_pad, 1), lambda c: (0, 0)),
        ],
        out_specs=(pl.BlockSpec((B_pad, 1), lambda c: (0, 0)),
                   pl.BlockSpec((B_pad, 1), lambda c: (0, 0))),
        scratch_shapes=[pltpu.VMEM((B_pad, D), jnp.bfloat16)],
        compiler_params=pltpu.CompilerParams(
            dimension_semantics=("arbitrary",),
            vmem_limit_bytes=100 * 1024 * 1024),
    )(embeddings, weight, labels2d)

    l = l_parts[:B, 0]
    T = t_parts[:B, 0]

    cos_m = math.cos(margin)
    sin_m = math.sin(margin)
    th = math.cos(math.pi - margin)
    mm = math.sin(math.pi - margin) * margin

    cos_t = jnp.log2(T) / s2
    sine = jnp.sqrt(jnp.clip(1.0 - cos_t * cos_t, 0.0, 1.0))
    phi = cos_t * cos_m - sine * sin_m
    phi = jnp.where(cos_t > th, phi, cos_t - mm)
    tl2 = phi * s2
    l_corr = (l - T) + jnp.exp2(tl2)
    per_row = (jnp.log2(l_corr) - tl2) * _LN2
    return jnp.mean(per_row)


def kernel(embeddings, weight, labels):
    return _arcface_loss(embeddings, weight, labels)


# speedup vs baseline: 1.0461x; 1.0461x over previous
import functools
import math

import jax
import jax.numpy as jnp
from jax import lax
from jax.experimental import pallas as pl
from jax.experimental.pallas import tpu as pltpu

_LOG2E = 1.4426950408889634
_LN2 = 0.6931471805599453


def _round_up(x, m):
    return (x + m - 1) // m * m


def _arcface_body(emb_ref, w_ref, lab_ref, l_ref, t_ref, embn_scr,
                  *, s2, num_classes, tile_c, nc, mask_cols):
    c = pl.program_id(0)

    @pl.when(c == 0)
    def _init():
        emb = emb_ref[...]
        inv = lax.rsqrt(jnp.maximum(jnp.sum(emb * emb, axis=1, keepdims=True),
                                    1e-24))
        embn_scr[...] = (emb * inv).astype(jnp.bfloat16)
        l_ref[...] = jnp.zeros(l_ref.shape, jnp.float32)
        t_ref[...] = jnp.zeros(t_ref.shape, jnp.float32)

    w = w_ref[...]
    inv_w = lax.rsqrt(jnp.maximum(jnp.sum(w * w, axis=1, keepdims=True), 1e-24))
    w_s = (w * (inv_w * s2)).astype(jnp.bfloat16)

    logits2 = lax.dot_general(
        w_s, embn_scr[...],
        dimension_numbers=(((1,), (1,)), ((), ())),
        preferred_element_type=jnp.float32)

    e = jnp.exp2(logits2)
    row = jax.lax.broadcasted_iota(jnp.int32, logits2.shape, 0)
    lab_loc = lab_ref[0:1, :] - c * tile_c
    one_hot = row == lab_loc
    if mask_cols:
        e = jnp.where(row + c * tile_c < num_classes, e, 0.0)
    l_ref[...] += jnp.sum(e, axis=0, keepdims=True)
    t_ref[...] += jnp.sum(jnp.where(one_hot, e, 0.0), axis=0, keepdims=True)


def _arcface_loss(embeddings, weight, labels, margin=0.2, scale=30.0):
    B, D = embeddings.shape
    C, D2 = weight.shape
    assert D == D2

    TILE_C = 4096
    B_pad = _round_up(B, 8)
    C_pad = _round_up(C, TILE_C)
    if B_pad != B:
        embeddings = jnp.pad(embeddings, ((0, B_pad - B), (0, 0)))
        labels = jnp.pad(labels, (0, B_pad - B))
    if C_pad != C:
        weight = jnp.pad(weight, ((0, C_pad - C), (0, 0)))
    nc = C_pad // TILE_C
    labels2d = jnp.broadcast_to(labels.astype(jnp.int32).reshape(1, B_pad),
                                (8, B_pad))

    s2 = scale * _LOG2E
    body = functools.partial(
        _arcface_body, s2=s2, num_classes=C, tile_c=TILE_C, nc=nc,
        mask_cols=(C_pad != C))

    l_parts, t_parts = pl.pallas_call(
        body,
        out_shape=(jax.ShapeDtypeStruct((1, B_pad), jnp.float32),
                   jax.ShapeDtypeStruct((1, B_pad), jnp.float32)),
        grid=(nc,),
        in_specs=[
            pl.BlockSpec((B_pad, D), lambda c: (0, 0)),
            pl.BlockSpec((TILE_C, D), lambda c: (c, 0)),
            pl.BlockSpec((8, B_pad), lambda c: (0, 0)),
        ],
        out_specs=(pl.BlockSpec((1, B_pad), lambda c: (0, 0)),
                   pl.BlockSpec((1, B_pad), lambda c: (0, 0))),
        scratch_shapes=[pltpu.VMEM((B_pad, D), jnp.bfloat16)],
        compiler_params=pltpu.CompilerParams(
            dimension_semantics=("arbitrary",),
            vmem_limit_bytes=100 * 1024 * 1024),
    )(embeddings, weight, labels2d)

    l = l_parts[0, :B]
    T = t_parts[0, :B]

    cos_m = math.cos(margin)
    sin_m = math.sin(margin)
    th = math.cos(math.pi - margin)
    mm = math.sin(math.pi - margin) * margin

    cos_t = jnp.log2(T) / s2
    sine = jnp.sqrt(jnp.clip(1.0 - cos_t * cos_t, 0.0, 1.0))
    phi = cos_t * cos_m - sine * sin_m
    phi = jnp.where(cos_t > th, phi, cos_t - mm)
    tl2 = phi * s2
    l_corr = (l - T) + jnp.exp2(tl2)
    per_row = (jnp.log2(l_corr) - tl2) * _LN2
    return jnp.mean(per_row)


def kernel(embeddings, weight, labels):
    return _arcface_loss(embeddings, weight, labels)
